# Initial kernel scaffold; baseline (speedup 1.0000x reference)
import jax
import jax.numpy as jnp
from jax.experimental import pallas as pl


def kernel(x, C):
    raise NotImplementedError("write your pallas kernel here")


# speedup vs baseline: 1.2848x; 1.2848x over previous
import functools

import jax
import jax.numpy as jnp
from jax import lax
from jax.experimental import pallas as pl
from jax.experimental.pallas import tpu as pltpu
from jax.experimental.pallas import tpu_sc as plsc

_NC = 2
_NS = 16
_NW = _NC * _NS
_G = 128
_K = 8


def kernel(x, C):
    B, H = x.shape
    V, D = C.shape
    N = B * H
    ngroups = N // _G
    gpw = ngroups // _NW
    nch = gpw // _K
    assert ngroups * _G == N and gpw * _NW == ngroups and nch * _K == gpw

    xg = x.reshape(ngroups, _G)

    mesh = plsc.VectorSubcoreMesh(
        core_axis_name="c", subcore_axis_name="s",
        num_cores=_NC, num_subcores=_NS)

    @functools.partial(
        pl.kernel,
        out_type=jax.ShapeDtypeStruct((ngroups, _G, D), jnp.float32),
        mesh=mesh,
        scratch_types=[
            pltpu.VMEM((_K, _G), jnp.int32),
            pltpu.VMEM((_K, _G, D), jnp.float32),
            pltpu.SemaphoreType.DMA,
        ],
        compiler_params=pltpu.CompilerParams(use_tc_tiling_on_sc=False),
    )
    def emb(x_hbm, C_hbm, out_hbm, idx_v, rows_v, gsem):
        wid = lax.axis_index("s") * _NC + lax.axis_index("c")
        gb = wid * gpw

        @pl.loop(0, nch)
        def chunk(c):
            base = gb + c * _K
            pltpu.sync_copy(x_hbm.at[pl.ds(base, _K)], idx_v)
            descs = [
                pltpu.async_copy(C_hbm.at[idx_v.at[j]], rows_v.at[j], gsem)
                for j in range(_K)
            ]
            for d in descs:
                d.wait()
            pltpu.sync_copy(rows_v, out_hbm.at[pl.ds(base, _K)])

    out = emb(xg, C)
    return out.reshape(B, H, D)


# speedup vs baseline: 1.3090x; 1.0188x over previous
import functools

import jax
import jax.numpy as jnp
from jax import lax
from jax.experimental import pallas as pl
from jax.experimental.pallas import tpu as pltpu
from jax.experimental.pallas import tpu_sc as plsc

_NC = 2
_NS = 16
_NW = _NC * _NS
_G = 128
_K = 8


def kernel(x, C):
    B, H = x.shape
    V, D = C.shape
    N = B * H
    ngroups = N // _G
    gpw = ngroups // _NW
    nch = gpw // _K
    assert ngroups * _G == N and gpw * _NW == ngroups and nch * _K == gpw
    assert nch >= 3 and nch % 2 == 1

    xg = x.reshape(ngroups, _G)

    mesh = plsc.VectorSubcoreMesh(
        core_axis_name="c", subcore_axis_name="s",
        num_cores=_NC, num_subcores=_NS)

    @functools.partial(
        pl.kernel,
        out_type=jax.ShapeDtypeStruct((ngroups, _G, D), jnp.float32),
        mesh=mesh,
        scratch_types=[
            pltpu.VMEM((2, _K, _G), jnp.int32),
            pltpu.VMEM((2, _K, _G, D), jnp.float32),
            pltpu.SemaphoreType.DMA,
            pltpu.SemaphoreType.DMA,
            pltpu.SemaphoreType.DMA,
            pltpu.SemaphoreType.DMA,
        ],
        compiler_params=pltpu.CompilerParams(use_tc_tiling_on_sc=False),
    )
    def emb(x_hbm, C_hbm, out_hbm, idx_v, rows_v, g0, g1, o0, o1):
        wid = lax.axis_index("s") * _NC + lax.axis_index("c")
        gb = wid * gpw
        gsem = (g0, g1)
        osem = (o0, o1)

        def load(c, s):
            pltpu.sync_copy(x_hbm.at[pl.ds(gb + c * _K, _K)], idx_v.at[s])

        def gfire(c, s):
            for j in range(_K):
                pltpu.async_copy(C_hbm.at[idx_v.at[s, j]], rows_v.at[s, j],
                                 gsem[s])

        def gdrain(s):
            pltpu.make_async_copy(
                out_hbm.at[pl.ds(0, _K)], rows_v.at[s], gsem[s]).wait()

        def sfire(c, s):
            pltpu.async_copy(rows_v.at[s], out_hbm.at[pl.ds(gb + c * _K, _K)],
                             osem[s])

        def sdrain(s):
            pltpu.make_async_copy(
                rows_v.at[s], out_hbm.at[pl.ds(0, _K)], osem[s]).wait()

        load(0, 0)
        gfire(0, 0)
        load(1, 1)
        gfire(1, 1)
        gdrain(0)
        sfire(0, 0)
        load(2, 0)
        sdrain(0)
        gfire(2, 0)
        gdrain(1)
        sfire(1, 1)

        @pl.loop(3, nch - 1, step=2)
        def pair(c):
            load(c, 1)
            sdrain(1)
            gfire(c, 1)
            gdrain(0)
            sfire(c - 1, 0)
            load(c + 1, 0)
            sdrain(0)
            gfire(c + 1, 0)
            gdrain(1)
            sfire(c, 1)

        gdrain(0)
        sfire(nch - 1, 0)
        sdrain(1)
        sdrain(0)

    out = emb(xg, C)
    return out.reshape(B, H, D)
